# baseline (speedup 1.0000x reference)
import functools

import jax
import jax.numpy as jnp
from jax import lax
from jax.experimental import pallas as pl
from jax.experimental.pallas import tpu as pltpu
from jax.experimental.pallas import tpu_sc as plsc

_K = 8192
_D = 256
_N = 16384
_BETA = 0.25

_TZ = 512
_CC = 1024
_TL = 2048
_GCH = 128


_WIN = (0, 2736, 5472, _K)

def _chunk_segments(c):
    segs = []
    clo, chi = c * _CC, (c + 1) * _CC
    for wdx in range(3):
        lo, hi = max(clo, _WIN[wdx]), min(chi, _WIN[wdx + 1])
        if lo < hi:
            segs.append((wdx, lo - clo, hi - clo))
    return segs


def _argmin_body(z_ref, w_ref, idx_ref, w2_ref, wbf_ref):
    i = pl.program_id(0)

    @pl.when(i == 0)
    def _compute_w2():
        ww = w_ref[...] * w_ref[...]
        ones = jnp.ones((1, _D), jnp.float32)
        w2 = lax.dot_general(
            ones, ww, (((1,), (1,)), ((), ())),
            precision=lax.Precision.HIGHEST,
            preferred_element_type=jnp.float32)
        gi = lax.broadcasted_iota(jnp.int32, (1, _K), 1)
        for wdx in range(3):
            w2_ref[wdx:wdx + 1, :] = jnp.where(
                (gi >= _WIN[wdx]) & (gi < _WIN[wdx + 1]), w2,
                jnp.float32(jnp.inf))
        wbf_ref[...] = w_ref[...].astype(jnp.bfloat16)

    zb = z_ref[...]
    z2 = jnp.sum(zb * zb, axis=1, keepdims=True)
    zm2 = (zb * jnp.float32(-2.0)).astype(jnp.bfloat16)

    inf = jnp.full((_TZ, 1), jnp.inf, jnp.float32)
    wvals = [inf, inf, inf]
    widxs = [jnp.full((_TZ, 1), jnp.float32(0.0))] * 3
    for c in range(_K // _CC):
        wc = wbf_ref[pl.ds(c * _CC, _CC), :]
        zw2 = lax.dot_general(
            zm2, wc, (((1,), (1,)), ((), ())),
            preferred_element_type=jnp.float32)
        t = z2 + zw2
        gidx = (lax.broadcasted_iota(jnp.int32, (_TZ, _CC), 1)
                .astype(jnp.float32) + jnp.float32(c * _CC))
        for wdx, _llo, _lhi in _chunk_segments(c):
            dseg = t + w2_ref[wdx:wdx + 1, pl.ds(c * _CC, _CC)]
            m = jnp.min(dseg, axis=1, keepdims=True)
            sidx = jnp.min(jnp.where(dseg == m, gidx, jnp.float32(_K)),
                           axis=1, keepdims=True)
            upd = m < wvals[wdx]
            wvals[wdx] = jnp.where(upd, m, wvals[wdx])
            widxs[wdx] = jnp.where(upd, sidx, widxs[wdx])

    run_v = inf
    run_i = jnp.full((_TZ, 1), jnp.float32(0.0))
    for wdx in range(3):
        lt = wvals[wdx] < run_v
        eq = wvals[wdx] == run_v
        take_i = lt | (eq & (widxs[wdx] < run_i))
        run_i = jnp.where(take_i, widxs[wdx], run_i)
        run_v = jnp.where(lt, wvals[wdx], run_v)
        run_v = run_v.astype(jnp.bfloat16).astype(jnp.float32)
    idx_ref[...] = run_i.astype(jnp.int32)


def _argmin_call(z_flat, weight):
    return pl.pallas_call(
        _argmin_body,
        grid=(_N // _TZ,),
        in_specs=[
            pl.BlockSpec((_TZ, _D), lambda i: (i, 0)),
            pl.BlockSpec((_K, _D), lambda i: (0, 0)),
        ],
        out_specs=pl.BlockSpec((_TZ, 1), lambda i: (i, 0)),
        out_shape=jax.ShapeDtypeStruct((_N, 1), jnp.int32),
        scratch_shapes=[pltpu.VMEM((3, _K), jnp.float32),
                        pltpu.VMEM((_K, _D), jnp.bfloat16)],
    )(z_flat, weight)


def _gather_call(idx, weight):
    mesh = plsc.VectorSubcoreMesh(core_axis_name="c", subcore_axis_name="s")
    per_worker = _N // 32

    @functools.partial(
        pl.kernel,
        mesh=mesh,
        out_type=jax.ShapeDtypeStruct((_N, _D), jnp.float32),
        scratch_types=[
            pltpu.VMEM((_GCH,), jnp.int32),
            pltpu.VMEM((_GCH, _D), jnp.float32),
            pltpu.SemaphoreType.DMA,
        ],
    )
    def gather_k(idx_hbm, table_hbm, out_hbm, idx_v, rows_v, sem):
        wid = lax.axis_index("s") * 2 + lax.axis_index("c")
        base = wid * per_worker
        for ch in range(per_worker // _GCH):
            o = base + ch * _GCH
            pltpu.sync_copy(idx_hbm.at[pl.ds(o, _GCH)], idx_v)
            pltpu.async_copy(table_hbm.at[idx_v], rows_v, sem).wait()
            pltpu.sync_copy(rows_v, out_hbm.at[pl.ds(o, _GCH)])

    return gather_k(idx, weight)


def _st_loss_body(zv_ref, zt_ref, q_ref, st_ref, loss_ref, acc_ref):
    i = pl.program_id(0)
    q = q_ref[...]
    zv = zv_ref[...]
    st_ref[...] = zv + (q - zv)
    d = q - zt_ref[...]
    s = jnp.sum(d * d)
    prev = jnp.where(i == 0, jnp.float32(0.0), acc_ref[0, 0])
    acc_ref[0, 0] = prev + s

    @pl.when(i == pl.num_programs(0) - 1)
    def _done():
        loss_ref[0, 0] = acc_ref[0, 0]


def _st_loss_call(z_view, z_flat, q):
    return pl.pallas_call(
        _st_loss_body,
        grid=(_N // _TL,),
        in_specs=[
            pl.BlockSpec((_TL, _D), lambda i: (i, 0)),
            pl.BlockSpec((_TL, _D), lambda i: (i, 0)),
            pl.BlockSpec((_TL, _D), lambda i: (i, 0)),
        ],
        out_specs=[
            pl.BlockSpec((_TL, _D), lambda i: (i, 0)),
            pl.BlockSpec(memory_space=pltpu.SMEM),
        ],
        out_shape=[
            jax.ShapeDtypeStruct((_N, _D), jnp.float32),
            jax.ShapeDtypeStruct((1, 1), jnp.float32),
        ],
        scratch_shapes=[pltpu.SMEM((1, 1), jnp.float32)],
    )(z_view, z_flat, q)


def kernel(z, weight):
    z_flat = jnp.transpose(z, (0, 2, 3, 1)).reshape(-1, _D)
    idx = _argmin_call(z_flat, weight).reshape(-1)
    q = _gather_call(idx, weight)
    z_view = z.reshape(-1, _D)
    st_flat, sq = _st_loss_call(z_view, z_flat, q)
    m = sq[0, 0] / jnp.float32(_N * _D)
    return st_flat.reshape(z.shape), _BETA * m, m


# speedup vs baseline: 1.1799x; 1.1799x over previous
import functools

import jax
import jax.numpy as jnp
from jax import lax
from jax.experimental import pallas as pl
from jax.experimental.pallas import tpu as pltpu
from jax.experimental.pallas import tpu_sc as plsc

_K = 8192
_D = 256
_N = 16384
_BETA = 0.25

_TZ = 512
_CC = 1024
_TL = 2048
_GCH = 128


_WIN = (0, 2736, 5472, _K)

def _chunk_segments(c):
    segs = []
    clo, chi = c * _CC, (c + 1) * _CC
    for wdx in range(3):
        lo, hi = max(clo, _WIN[wdx]), min(chi, _WIN[wdx + 1])
        if lo < hi:
            segs.append((wdx, lo - clo, hi - clo))
    return segs


def _argmin_body(z_ref, w_ref, idx_ref, loss_ref, w2_ref, wbf_ref, acc_ref):
    i = pl.program_id(0)

    @pl.when(i == 0)
    def _compute_w2():
        ww = w_ref[...] * w_ref[...]
        ones = jnp.ones((1, _D), jnp.float32)
        w2 = lax.dot_general(
            ones, ww, (((1,), (1,)), ((), ())),
            precision=lax.Precision.HIGHEST,
            preferred_element_type=jnp.float32)
        gi = lax.broadcasted_iota(jnp.int32, (1, _K), 1)
        for wdx in range(3):
            w2_ref[wdx:wdx + 1, :] = jnp.where(
                (gi >= _WIN[wdx]) & (gi < _WIN[wdx + 1]), w2,
                jnp.float32(jnp.inf))
        wbf_ref[...] = w_ref[...].astype(jnp.bfloat16)

    zb = z_ref[...]
    z2 = jnp.sum(zb * zb, axis=1, keepdims=True)
    zm2 = (zb * jnp.float32(-2.0)).astype(jnp.bfloat16)

    inf = jnp.full((_TZ, 1), jnp.inf, jnp.float32)
    wvals = [inf, inf, inf]
    widxs = [jnp.full((_TZ, 1), jnp.float32(0.0))] * 3
    for c in range(_K // _CC):
        wc = wbf_ref[pl.ds(c * _CC, _CC), :]
        zw2 = lax.dot_general(
            zm2, wc, (((1,), (1,)), ((), ())),
            preferred_element_type=jnp.float32)
        t = z2 + zw2
        gidx = (lax.broadcasted_iota(jnp.int32, (_TZ, _CC), 1)
                .astype(jnp.float32) + jnp.float32(c * _CC))
        for wdx, _llo, _lhi in _chunk_segments(c):
            dseg = t + w2_ref[wdx:wdx + 1, pl.ds(c * _CC, _CC)]
            m = jnp.min(dseg, axis=1, keepdims=True)
            sidx = jnp.min(jnp.where(dseg == m, gidx, jnp.float32(_K)),
                           axis=1, keepdims=True)
            upd = m < wvals[wdx]
            wvals[wdx] = jnp.where(upd, m, wvals[wdx])
            widxs[wdx] = jnp.where(upd, sidx, widxs[wdx])

    run_v = inf
    run_i = jnp.full((_TZ, 1), jnp.float32(0.0))
    for wdx in range(3):
        lt = wvals[wdx] < run_v
        eq = wvals[wdx] == run_v
        take_i = lt | (eq & (widxs[wdx] < run_i))
        run_i = jnp.where(take_i, widxs[wdx], run_i)
        run_v = jnp.where(lt, wvals[wdx], run_v)
        run_v = run_v.astype(jnp.bfloat16).astype(jnp.float32)
    idx_ref[...] = run_i.astype(jnp.int32)

    dmin = jnp.minimum(jnp.minimum(wvals[0], wvals[1]), wvals[2])
    part = jnp.sum(dmin)
    prev = jnp.where(i == 0, jnp.float32(0.0), acc_ref[0, 0])
    acc_ref[0, 0] = prev + part

    @pl.when(i == pl.num_programs(0) - 1)
    def _done():
        loss_ref[0, 0] = acc_ref[0, 0]


def _argmin_call(z_flat, weight):
    return pl.pallas_call(
        _argmin_body,
        grid=(_N // _TZ,),
        in_specs=[
            pl.BlockSpec((_TZ, _D), lambda i: (i, 0)),
            pl.BlockSpec((_K, _D), lambda i: (0, 0)),
        ],
        out_specs=[
            pl.BlockSpec((_TZ, 1), lambda i: (i, 0)),
            pl.BlockSpec(memory_space=pltpu.SMEM),
        ],
        out_shape=[
            jax.ShapeDtypeStruct((_N, 1), jnp.int32),
            jax.ShapeDtypeStruct((1, 1), jnp.float32),
        ],
        scratch_shapes=[pltpu.VMEM((3, _K), jnp.float32),
                        pltpu.VMEM((_K, _D), jnp.bfloat16),
                        pltpu.SMEM((1, 1), jnp.float32)],
    )(z_flat, weight)


def _gather_call(idx, weight):
    mesh = plsc.VectorSubcoreMesh(core_axis_name="c", subcore_axis_name="s")
    per_worker = _N // 32

    @functools.partial(
        pl.kernel,
        mesh=mesh,
        out_type=jax.ShapeDtypeStruct((_N, _D), jnp.float32),
        scratch_types=[
            pltpu.VMEM((_GCH,), jnp.int32),
            pltpu.VMEM((_GCH, _D), jnp.float32),
            pltpu.SemaphoreType.DMA,
        ],
    )
    def gather_k(idx_hbm, table_hbm, out_hbm, idx_v, rows_v, sem):
        wid = lax.axis_index("s") * 2 + lax.axis_index("c")
        base = wid * per_worker
        for ch in range(per_worker // _GCH):
            o = base + ch * _GCH
            pltpu.sync_copy(idx_hbm.at[pl.ds(o, _GCH)], idx_v)
            pltpu.async_copy(table_hbm.at[idx_v], rows_v, sem).wait()
            pltpu.sync_copy(rows_v, out_hbm.at[pl.ds(o, _GCH)])

    return gather_k(idx, weight)


def kernel(z, weight):
    z_flat = jnp.transpose(z, (0, 2, 3, 1)).reshape(-1, _D)
    idx2, sq = _argmin_call(z_flat, weight)
    q = _gather_call(idx2.reshape(-1), weight)
    m = sq[0, 0] / jnp.float32(_N * _D)
    return q.reshape(z.shape), _BETA * m, m


# speedup vs baseline: 1.2923x; 1.0953x over previous
import functools

import jax
import jax.numpy as jnp
from jax import lax
from jax.experimental import pallas as pl
from jax.experimental.pallas import tpu as pltpu
from jax.experimental.pallas import tpu_sc as plsc

_K = 8192
_D = 256
_N = 16384
_BETA = 0.25

_TZ = 512
_CC = 1024
_TL = 2048
_GCH = 128


_WIN = (0, 2736, 5472, _K)
_WLEN = (2736, 2736, 2720)
_CCW = 2752


def _argmin_body(z_ref, w_ref, idx_ref, loss_ref, w2_ref, wbf_ref, acc_ref):
    i = pl.program_id(0)

    @pl.when(i == 0)
    def _compute_w2():
        ones = jnp.ones((1, _D), jnp.float32)
        w2_ref[...] = jnp.full((3, _CCW), jnp.inf, jnp.float32)
        wbf_ref[...] = jnp.zeros((3 * _CCW, _D), jnp.bfloat16)
        for wdx in range(3):
            lo, wl = _WIN[wdx], _WLEN[wdx]
            wseg = w_ref[pl.ds(lo, wl), :]
            w2_ref[wdx:wdx + 1, 0:wl] = lax.dot_general(
                ones, wseg * wseg, (((1,), (1,)), ((), ())),
                precision=lax.Precision.HIGHEST,
                preferred_element_type=jnp.float32)
            wbf_ref[pl.ds(wdx * _CCW, wl), :] = wseg.astype(jnp.bfloat16)

    zb = z_ref[...]
    z2 = jnp.sum(zb * zb, axis=1, keepdims=True)
    zm2 = (zb * jnp.float32(-2.0)).astype(jnp.bfloat16)

    wvals = [None] * 3
    widxs = [None] * 3
    for wdx in range(3):
        wc = wbf_ref[pl.ds(wdx * _CCW, _CCW), :]
        zw2 = lax.dot_general(
            zm2, wc, (((1,), (1,)), ((), ())),
            preferred_element_type=jnp.float32)
        dseg = (z2 + zw2) + w2_ref[wdx:wdx + 1, :]
        gidx = (lax.broadcasted_iota(jnp.int32, (_TZ, _CCW), 1)
                .astype(jnp.float32) + jnp.float32(_WIN[wdx]))
        m = jnp.min(dseg, axis=1, keepdims=True)
        wvals[wdx] = m
        widxs[wdx] = jnp.min(jnp.where(dseg == m, gidx, jnp.float32(_K)),
                             axis=1, keepdims=True)

    run_v = jnp.full((_TZ, 1), jnp.inf, jnp.float32)
    run_i = jnp.full((_TZ, 1), jnp.float32(0.0))
    for wdx in range(3):
        lt = wvals[wdx] < run_v
        eq = wvals[wdx] == run_v
        take_i = lt | (eq & (widxs[wdx] < run_i))
        run_i = jnp.where(take_i, widxs[wdx], run_i)
        run_v = jnp.where(lt, wvals[wdx], run_v)
        run_v = run_v.astype(jnp.bfloat16).astype(jnp.float32)
    idx_ref[...] = run_i.astype(jnp.int32)

    dmin = jnp.minimum(jnp.minimum(wvals[0], wvals[1]), wvals[2])
    part = jnp.sum(dmin)
    prev = jnp.where(i == 0, jnp.float32(0.0), acc_ref[0, 0])
    acc_ref[0, 0] = prev + part

    @pl.when(i == pl.num_programs(0) - 1)
    def _done():
        loss_ref[0, 0] = acc_ref[0, 0]


def _argmin_call(z_flat, weight):
    return pl.pallas_call(
        _argmin_body,
        grid=(_N // _TZ,),
        in_specs=[
            pl.BlockSpec((_TZ, _D), lambda i: (i, 0)),
            pl.BlockSpec((_K, _D), lambda i: (0, 0)),
        ],
        out_specs=[
            pl.BlockSpec((_TZ, 1), lambda i: (i, 0)),
            pl.BlockSpec(memory_space=pltpu.SMEM),
        ],
        out_shape=[
            jax.ShapeDtypeStruct((_N, 1), jnp.int32),
            jax.ShapeDtypeStruct((1, 1), jnp.float32),
        ],
        scratch_shapes=[pltpu.VMEM((3, _CCW), jnp.float32),
                        pltpu.VMEM((3 * _CCW, _D), jnp.bfloat16),
                        pltpu.SMEM((1, 1), jnp.float32)],
    )(z_flat, weight)


def _gather_call(idx, weight):
    mesh = plsc.VectorSubcoreMesh(core_axis_name="c", subcore_axis_name="s")
    per_worker = _N // 32

    @functools.partial(
        pl.kernel,
        mesh=mesh,
        out_type=jax.ShapeDtypeStruct((_N, _D), jnp.float32),
        scratch_types=[
            pltpu.VMEM((_GCH,), jnp.int32),
            pltpu.VMEM((_GCH, _D), jnp.float32),
            pltpu.SemaphoreType.DMA,
        ],
    )
    def gather_k(idx_hbm, table_hbm, out_hbm, idx_v, rows_v, sem):
        wid = lax.axis_index("s") * 2 + lax.axis_index("c")
        base = wid * per_worker
        for ch in range(per_worker // _GCH):
            o = base + ch * _GCH
            pltpu.sync_copy(idx_hbm.at[pl.ds(o, _GCH)], idx_v)
            pltpu.async_copy(table_hbm.at[idx_v], rows_v, sem).wait()
            pltpu.sync_copy(rows_v, out_hbm.at[pl.ds(o, _GCH)])

    return gather_k(idx, weight)


def kernel(z, weight):
    z_flat = jnp.transpose(z, (0, 2, 3, 1)).reshape(-1, _D)
    idx2, sq = _argmin_call(z_flat, weight)
    q = _gather_call(idx2.reshape(-1), weight)
    m = sq[0, 0] / jnp.float32(_N * _D)
    return q.reshape(z.shape), _BETA * m, m
